# Initial kernel scaffold; baseline (speedup 1.0000x reference)
import jax
import jax.numpy as jnp
from jax.experimental import pallas as pl


def kernel(logits, uniform):
    raise NotImplementedError("write your pallas kernel here")


# speedup vs baseline: 2.2367x; 2.2367x over previous
import numpy as np
import jax
import jax.numpy as jnp
from jax.experimental import pallas as pl
from jax.experimental.pallas import tpu as pltpu

TAU_INV = 10.0
KSEL = 8


def _row_body(logits_ref, logits_r_ref, u_ref, samples_ref, discrete_ref,
              noisy_ref):
    kk, d = noisy_ref.shape
    ch = min(2048, d)
    nch = d // ch
    tiny = np.float32(np.finfo(np.float32).tiny)
    neg_inf = np.float32(-np.inf)

    def phase_a(j, m):
        js = pl.ds(j * ch, ch)
        u = u_ref[0, :, js]
        w = -jnp.log(jnp.clip(u, tiny, 1.0))
        nl = (-jnp.log(w) + logits_ref[0, :, js]) * TAU_INV
        noisy_ref[:, js] = nl
        return jnp.maximum(m, jnp.max(nl, axis=1, keepdims=True))

    m = jax.lax.fori_loop(
        0, nch, phase_a, jnp.full((kk, 1), neg_inf, jnp.float32))

    def phase_a2(j, s):
        js = pl.ds(j * ch, ch)
        return s + jnp.sum(jnp.exp(noisy_ref[:, js] - m), axis=1,
                           keepdims=True)

    s = jax.lax.fori_loop(
        0, nch, phase_a2, jnp.zeros((kk, 1), jnp.float32))
    log_s = jnp.log(s)

    def phase_b(j, carry):
        js = pl.ds(j * ch, ch)
        y = (noisy_ref[:, js] - m) - log_s
        ym = jnp.max(y, axis=0, keepdims=True)
        samples_ref[0, :, js] = jnp.exp(ym)
        return carry

    jax.lax.fori_loop(0, nch, phase_b, jnp.int32(0))

    lr = logits_r_ref[0]

    def phase_c(i, carry):
        t, thr, cum, done = carry
        cand = jnp.where(lr < t, lr, neg_inf)
        mx = jnp.max(cand)
        cnt = jnp.sum(jnp.where(lr == mx, 1.0, 0.0))
        cum = cum + cnt
        hit = cum >= np.float32(KSEL)
        newly = jnp.logical_and(jnp.logical_not(done), hit)
        thr = jnp.where(newly, mx, thr)
        done = jnp.logical_or(done, hit)
        return (mx, thr, cum, done)

    init = (jnp.float32(np.inf), jnp.float32(0.0), jnp.float32(0.0),
            jnp.bool_(False))
    _, thr, _, _ = jax.lax.fori_loop(0, KSEL, phase_c, init)
    discrete_ref[0] = jnp.where(lr >= thr, 1.0, 0.0).astype(jnp.float32)


def _build(B, K, d, interpret=False):
    rd = d // 8
    return pl.pallas_call(
        _row_body,
        grid=(B,),
        in_specs=[
            pl.BlockSpec((1, 1, d), lambda i: (i, 0, 0)),
            pl.BlockSpec((1, 8, rd), lambda i: (i, 0, 0)),
            pl.BlockSpec((1, K, d), lambda i: (i, 0, 0)),
        ],
        out_specs=[
            pl.BlockSpec((1, 1, d), lambda i: (i, 0, 0)),
            pl.BlockSpec((1, 8, rd), lambda i: (i, 0, 0)),
        ],
        out_shape=[
            jax.ShapeDtypeStruct((B, 1, d), jnp.float32),
            jax.ShapeDtypeStruct((B, 8, rd), jnp.float32),
        ],
        scratch_shapes=[pltpu.VMEM((K, d), jnp.float32)],
        compiler_params=pltpu.CompilerParams(
            dimension_semantics=("arbitrary",)),
        interpret=interpret,
    )


def kernel(logits, uniform):
    B, d = logits.shape
    K = uniform.shape[1]
    logits_r = logits.reshape(B, 8, d // 8)
    samples, discrete_r = _build(B, K, d)(
        logits.reshape(B, 1, d), logits_r, uniform)
    return samples.reshape(B, d), discrete_r.reshape(B, d)


# speedup vs baseline: 8.8945x; 3.9766x over previous
import numpy as np
import jax
import jax.numpy as jnp
from jax.experimental import pallas as pl
from jax.experimental.pallas import tpu as pltpu

TAU_INV = 10.0
KSEL = 8
RB = 8


def _block_body(logits_ref, u_ref, samples_ref, discrete_ref, noisy_ref):
    _, kk, d = noisy_ref.shape
    ch = min(2048, d)
    nch = d // ch
    nsl = ch // 128
    tiny = np.float32(np.finfo(np.float32).tiny)
    neg_inf = np.float32(-np.inf)

    def phase_a(j, carry):
        ms, ss = carry
        js = pl.ds(j * ch, ch)
        new_ms, new_ss = [], []
        for r in range(RB):
            u = u_ref[r, :, js]
            w = -jnp.log(jnp.clip(u, tiny, 1.0))
            nl = (-jnp.log(w) + logits_ref[pl.ds(r, 1), js]) * TAU_INV
            noisy_ref[r, :, js] = nl
            cm = nl[:, 0:128]
            for s in range(1, nsl):
                cm = jnp.maximum(cm, nl[:, s * 128:(s + 1) * 128])
            mn = jnp.maximum(ms[r], cm)
            se = jnp.exp(nl[:, 0:128] - mn)
            for s in range(1, nsl):
                se = se + jnp.exp(nl[:, s * 128:(s + 1) * 128] - mn)
            new_ss.append(ss[r] * jnp.exp(ms[r] - mn) + se)
            new_ms.append(mn)
        return tuple(new_ms), tuple(new_ss)

    init_m = tuple(jnp.full((kk, 128), neg_inf, jnp.float32)
                   for _ in range(RB))
    init_s = tuple(jnp.zeros((kk, 128), jnp.float32) for _ in range(RB))
    ms, ss = jax.lax.fori_loop(0, nch, phase_a, (init_m, init_s))

    cs = []
    for r in range(RB):
        m = jnp.max(ms[r], axis=1, keepdims=True)
        s = jnp.sum(ss[r] * jnp.exp(ms[r] - m), axis=1, keepdims=True)
        cs.append(m + jnp.log(s))

    def phase_b(j, carry):
        js = pl.ds(j * ch, ch)
        rows = []
        for r in range(RB):
            y = noisy_ref[r, :, js] - cs[r]
            rows.append(jnp.max(y, axis=0, keepdims=True))
        stacked = jnp.concatenate(rows, axis=0)
        samples_ref[:, js] = jnp.exp(stacked)
        return carry

    jax.lax.fori_loop(0, nch, phase_b, jnp.int32(0))

    lr = logits_ref[...]

    def phase_c(i, carry):
        t, thr, cum, done = carry
        cand = jnp.where(lr < t, lr, neg_inf)
        mx = jnp.max(cand, axis=1, keepdims=True)
        cnt = jnp.sum(jnp.where(lr == mx, 1.0, 0.0), axis=1,
                      keepdims=True)
        cum = cum + cnt
        hit = jnp.where(cum >= np.float32(KSEL), 1.0, 0.0)
        newly = hit * (1.0 - done)
        thr = jnp.where(newly > 0.0, mx, thr)
        done = jnp.maximum(done, hit)
        return (mx, thr, cum, done)

    init = (jnp.full((RB, 1), np.inf, jnp.float32),
            jnp.zeros((RB, 1), jnp.float32),
            jnp.zeros((RB, 1), jnp.float32),
            jnp.zeros((RB, 1), jnp.float32))
    _, thr, _, _ = jax.lax.fori_loop(0, KSEL, phase_c, init)
    discrete_ref[...] = jnp.where(lr >= thr, 1.0, 0.0).astype(jnp.float32)


def _build(B, K, d, interpret=False):
    return pl.pallas_call(
        _block_body,
        grid=(B // RB,),
        in_specs=[
            pl.BlockSpec((RB, d), lambda i: (i, 0)),
            pl.BlockSpec((RB, K, d), lambda i: (i, 0, 0)),
        ],
        out_specs=[
            pl.BlockSpec((RB, d), lambda i: (i, 0)),
            pl.BlockSpec((RB, d), lambda i: (i, 0)),
        ],
        out_shape=[
            jax.ShapeDtypeStruct((B, d), jnp.float32),
            jax.ShapeDtypeStruct((B, d), jnp.float32),
        ],
        scratch_shapes=[pltpu.VMEM((RB, K, d), jnp.float32)],
        compiler_params=pltpu.CompilerParams(
            dimension_semantics=("arbitrary",)),
        interpret=interpret,
    )


def kernel(logits, uniform):
    B, d = logits.shape
    K = uniform.shape[1]
    samples, discrete = _build(B, K, d)(logits, uniform)
    return samples, discrete


# speedup vs baseline: 11.1063x; 1.2487x over previous
import numpy as np
import jax
import jax.numpy as jnp
from jax.experimental import pallas as pl
from jax.experimental.pallas import tpu as pltpu

KSEL = 8
RB = 8
CA = np.float32(10.0 * np.log2(np.e))
CB = np.float32(-10.0 * np.log2(np.log(2.0)))
C10 = np.float32(10.0)


def _block_body(logits_ref, u_ref, samples_ref, discrete_ref, z_ref):
    _, kk, d = z_ref.shape
    ch = min(2048, d)
    nch = d // ch
    nsl = ch // 128
    tiny = np.float32(np.finfo(np.float32).tiny)
    neg_inf = np.float32(-np.inf)

    def phase_a(j, carry):
        ms, ss = carry
        js = pl.ds(j * ch, ch)
        new_ms, new_ss = [], []
        for r in range(RB):
            u = u_ref[r, :, js]
            wp = -jnp.log2(jnp.maximum(u, tiny))
            lp = CA * logits_ref[pl.ds(r, 1), js] + CB
            z = lp - C10 * jnp.log2(wp)
            z_ref[r, :, js] = z
            cm = z[:, 0:128]
            for s in range(1, nsl):
                cm = jnp.maximum(cm, z[:, s * 128:(s + 1) * 128])
            mn = jnp.maximum(ms[r], cm)
            se = jnp.exp2(z[:, 0:128] - mn)
            for s in range(1, nsl):
                se = se + jnp.exp2(z[:, s * 128:(s + 1) * 128] - mn)
            new_ss.append(ss[r] * jnp.exp2(ms[r] - mn) + se)
            new_ms.append(mn)
        return tuple(new_ms), tuple(new_ss)

    init_m = tuple(jnp.full((kk, 128), neg_inf, jnp.float32)
                   for _ in range(RB))
    init_s = tuple(jnp.zeros((kk, 128), jnp.float32) for _ in range(RB))
    ms, ss = jax.lax.fori_loop(0, nch, phase_a, (init_m, init_s))

    cs = []
    for r in range(RB):
        m = jnp.max(ms[r], axis=1, keepdims=True)
        s = jnp.sum(ss[r] * jnp.exp2(ms[r] - m), axis=1, keepdims=True)
        cs.append(m + jnp.log2(s))

    def phase_b(j, carry):
        js = pl.ds(j * ch, ch)
        rows = []
        for r in range(RB):
            y = z_ref[r, :, js] - cs[r]
            rows.append(jnp.max(y, axis=0, keepdims=True))
        stacked = jnp.concatenate(rows, axis=0)
        samples_ref[:, js] = jnp.exp2(stacked)
        return carry

    jax.lax.fori_loop(0, nch, phase_b, jnp.int32(0))

    lr = logits_ref[...]
    nb = min(4096, d)

    def iter8(arr):
        def body(i, carry):
            t, thr, cum, done = carry
            cand = jnp.where(arr < t, arr, neg_inf)
            mx = jnp.max(cand, axis=1, keepdims=True)
            cnt = jnp.sum(jnp.where(arr == mx, 1.0, 0.0), axis=1,
                          keepdims=True)
            cum = cum + cnt
            hit = jnp.where(cum >= np.float32(KSEL), 1.0, 0.0)
            newly = hit * (1.0 - done)
            thr = jnp.where(newly > 0.0, mx, thr)
            done = jnp.maximum(done, hit)
            return (mx, thr, cum, done)

        init = (jnp.full((RB, 1), np.inf, jnp.float32),
                jnp.zeros((RB, 1), jnp.float32),
                jnp.zeros((RB, 1), jnp.float32),
                jnp.zeros((RB, 1), jnp.float32))
        return jax.lax.fori_loop(0, KSEL, body, init)[1]

    p = lr[:, 0:nb]
    for s in range(1, d // nb):
        p = jnp.maximum(p, lr[:, s * nb:(s + 1) * nb])
    t_p = iter8(p)
    c_full = jnp.sum(jnp.where(lr >= t_p, 1.0, 0.0), axis=1,
                     keepdims=True)
    exact = jnp.all(c_full == np.float32(KSEL))
    thr = jax.lax.cond(exact, lambda: t_p, lambda: iter8(lr))
    discrete_ref[...] = jnp.where(lr >= thr, 1.0, 0.0).astype(jnp.float32)


def _build(B, K, d, interpret=False):
    return pl.pallas_call(
        _block_body,
        grid=(B // RB,),
        in_specs=[
            pl.BlockSpec((RB, d), lambda i: (i, 0)),
            pl.BlockSpec((RB, K, d), lambda i: (i, 0, 0)),
        ],
        out_specs=[
            pl.BlockSpec((RB, d), lambda i: (i, 0)),
            pl.BlockSpec((RB, d), lambda i: (i, 0)),
        ],
        out_shape=[
            jax.ShapeDtypeStruct((B, d), jnp.float32),
            jax.ShapeDtypeStruct((B, d), jnp.float32),
        ],
        scratch_shapes=[pltpu.VMEM((RB, K, d), jnp.float32)],
        compiler_params=pltpu.CompilerParams(
            dimension_semantics=("arbitrary",)),
        interpret=interpret,
    )


def kernel(logits, uniform):
    B, d = logits.shape
    K = uniform.shape[1]
    samples, discrete = _build(B, K, d)(logits, uniform)
    return samples, discrete


# speedup vs baseline: 11.4557x; 1.0315x over previous
import numpy as np
import jax
import jax.numpy as jnp
from jax.experimental import pallas as pl
from jax.experimental.pallas import tpu as pltpu

KSEL = 8
RB = 8
CA = np.float32(10.0 * np.log2(np.e))
CB = np.float32(-10.0 * np.log2(np.log(2.0)))
C10 = np.float32(10.0)


def _block_body(logits_ref, u_ref, samples_ref, discrete_ref, z_ref):
    _, kk, d = z_ref.shape
    ch = min(4096, d)
    nch = d // ch
    nsl = ch // 128
    tiny = np.float32(np.finfo(np.float32).tiny)
    neg_inf = np.float32(-np.inf)

    def phase_a(j, carry):
        ms, ss = carry
        js = pl.ds(j * ch, ch)
        new_ms, new_ss = [], []
        for r in range(RB):
            u = u_ref[r, :, js]
            wp = -jnp.log2(jnp.maximum(u, tiny))
            lp = CA * logits_ref[pl.ds(r, 1), js] + CB
            z = lp - C10 * jnp.log2(wp)
            z_ref[r, :, js] = z
            cm = z[:, 0:128]
            for s in range(1, nsl):
                cm = jnp.maximum(cm, z[:, s * 128:(s + 1) * 128])
            mn = jnp.maximum(ms[r], cm)
            se = jnp.exp2(z[:, 0:128] - mn)
            for s in range(1, nsl):
                se = se + jnp.exp2(z[:, s * 128:(s + 1) * 128] - mn)
            new_ss.append(ss[r] * jnp.exp2(ms[r] - mn) + se)
            new_ms.append(mn)
        return tuple(new_ms), tuple(new_ss)

    init_m = tuple(jnp.full((kk, 128), neg_inf, jnp.float32)
                   for _ in range(RB))
    init_s = tuple(jnp.zeros((kk, 128), jnp.float32) for _ in range(RB))
    ms, ss = jax.lax.fori_loop(0, nch, phase_a, (init_m, init_s))

    cs = []
    for r in range(RB):
        m = jnp.max(ms[r], axis=1, keepdims=True)
        s = jnp.sum(ss[r] * jnp.exp2(ms[r] - m), axis=1, keepdims=True)
        cs.append(m + jnp.log2(s))

    def phase_b(j, carry):
        js = pl.ds(j * ch, ch)
        rows = []
        for r in range(RB):
            y = z_ref[r, :, js] - cs[r]
            rows.append(jnp.max(y, axis=0, keepdims=True))
        stacked = jnp.concatenate(rows, axis=0)
        samples_ref[:, js] = jnp.exp2(stacked)
        return carry

    jax.lax.fori_loop(0, nch, phase_b, jnp.int32(0))

    lr = logits_ref[...]
    nb = min(4096, d)

    def iter8(arr):
        def body(i, carry):
            t, thr, cum, done = carry
            cand = jnp.where(arr < t, arr, neg_inf)
            mx = jnp.max(cand, axis=1, keepdims=True)
            cnt = jnp.sum(jnp.where(arr == mx, 1.0, 0.0), axis=1,
                          keepdims=True)
            cum = cum + cnt
            hit = jnp.where(cum >= np.float32(KSEL), 1.0, 0.0)
            newly = hit * (1.0 - done)
            thr = jnp.where(newly > 0.0, mx, thr)
            done = jnp.maximum(done, hit)
            return (mx, thr, cum, done)

        init = (jnp.full((RB, 1), np.inf, jnp.float32),
                jnp.zeros((RB, 1), jnp.float32),
                jnp.zeros((RB, 1), jnp.float32),
                jnp.zeros((RB, 1), jnp.float32))
        return jax.lax.fori_loop(0, KSEL, body, init)[1]

    p = lr[:, 0:nb]
    for s in range(1, d // nb):
        p = jnp.maximum(p, lr[:, s * nb:(s + 1) * nb])
    t_p = iter8(p)
    c_full = jnp.sum(jnp.where(lr >= t_p, 1.0, 0.0), axis=1,
                     keepdims=True)
    exact = jnp.all(c_full == np.float32(KSEL))
    thr = jax.lax.cond(exact, lambda: t_p, lambda: iter8(lr))
    discrete_ref[...] = jnp.where(lr >= thr, 1.0, 0.0).astype(jnp.float32)


def _build(B, K, d, interpret=False):
    return pl.pallas_call(
        _block_body,
        grid=(B // RB,),
        in_specs=[
            pl.BlockSpec((RB, d), lambda i: (i, 0)),
            pl.BlockSpec((RB, K, d), lambda i: (i, 0, 0)),
        ],
        out_specs=[
            pl.BlockSpec((RB, d), lambda i: (i, 0)),
            pl.BlockSpec((RB, d), lambda i: (i, 0)),
        ],
        out_shape=[
            jax.ShapeDtypeStruct((B, d), jnp.float32),
            jax.ShapeDtypeStruct((B, d), jnp.float32),
        ],
        scratch_shapes=[pltpu.VMEM((RB, K, d), jnp.float32)],
        compiler_params=pltpu.CompilerParams(
            dimension_semantics=("arbitrary",)),
        interpret=interpret,
    )


def kernel(logits, uniform):
    B, d = logits.shape
    K = uniform.shape[1]
    samples, discrete = _build(B, K, d)(logits, uniform)
    return samples, discrete


# speedup vs baseline: 12.2544x; 1.0697x over previous
import numpy as np
import jax
import jax.numpy as jnp
from jax.experimental import pallas as pl
from jax.experimental.pallas import tpu as pltpu

KSEL = 8
RB = 8
CA = np.float32(10.0 * np.log2(np.e))
CB = np.float32(-10.0 * np.log2(np.log(2.0)))
C10 = np.float32(10.0)


def _tree(op, items):
    while len(items) > 1:
        nxt = [op(items[i], items[i + 1]) for i in range(0, len(items) - 1, 2)]
        if len(items) % 2:
            nxt.append(items[-1])
        items = nxt
    return items[0]


def _block_body(logits_ref, u_ref, samples_ref, discrete_ref, z_ref, p_ref):
    _, kk, d = z_ref.shape
    ch = min(4096, d)
    nch = d // ch
    nsl = ch // 128
    tiny = np.float32(np.finfo(np.float32).tiny)
    neg_inf = np.float32(-np.inf)

    def phase_a(j, carry):
        ms, ss = carry
        js = pl.ds(j * ch, ch)
        new_ms, new_ss = [], []
        for r in range(RB):
            u = u_ref[r, :, js]
            wp = -jnp.log2(jnp.maximum(u, tiny))
            lp = CA * logits_ref[pl.ds(r, 1), js] + CB
            z = lp - C10 * jnp.log2(wp)
            z_ref[r, :, js] = z
            sl = [z[:, s * 128:(s + 1) * 128] for s in range(nsl)]
            cm = _tree(jnp.maximum, sl)
            mn = jnp.maximum(ms[r], cm)
            se = _tree(jnp.add, [jnp.exp2(x - mn) for x in sl])
            new_ss.append(ss[r] * jnp.exp2(ms[r] - mn) + se)
            new_ms.append(mn)
        return tuple(new_ms), tuple(new_ss)

    init_m = tuple(jnp.full((kk, 128), neg_inf, jnp.float32)
                   for _ in range(RB))
    init_s = tuple(jnp.zeros((kk, 128), jnp.float32) for _ in range(RB))
    ms, ss = jax.lax.fori_loop(0, nch, phase_a, (init_m, init_s))

    cs = []
    for r in range(RB):
        m = jnp.max(ms[r], axis=1, keepdims=True)
        s = jnp.sum(ss[r] * jnp.exp2(ms[r] - m), axis=1, keepdims=True)
        cs.append(m + jnp.log2(s))

    lr = logits_ref[...]
    nb = min(4096, d)
    p = _tree(jnp.maximum,
              [lr[:, s * nb:(s + 1) * nb] for s in range(d // nb)])
    p_ref[...] = p

    def topk_step(arr, carry):
        t, thr, cum, done = carry
        cand = jnp.where(arr < t, arr, neg_inf)
        mx = jnp.max(cand, axis=1, keepdims=True)
        cnt = jnp.sum(jnp.where(arr == mx, 1.0, 0.0), axis=1,
                      keepdims=True)
        cum = cum + cnt
        hit = jnp.where(cum >= np.float32(KSEL), 1.0, 0.0)
        newly = hit * (1.0 - done)
        thr = jnp.where(newly > 0.0, mx, thr)
        done = jnp.maximum(done, hit)
        return (mx, thr, cum, done)

    def topk_init():
        return (jnp.full((RB, 1), np.inf, jnp.float32),
                jnp.zeros((RB, 1), jnp.float32),
                jnp.zeros((RB, 1), jnp.float32),
                jnp.zeros((RB, 1), jnp.float32))

    fused = (nch == KSEL)

    def phase_b(j, carry):
        js = pl.ds(j * ch, ch)
        rows = []
        for r in range(RB):
            y = z_ref[r, :, js] - cs[r]
            rows.append(jnp.max(y, axis=0, keepdims=True))
        stacked = jnp.concatenate(rows, axis=0)
        samples_ref[:, js] = jnp.exp2(stacked)
        if fused:
            return topk_step(p_ref[...], carry)
        return carry

    if fused:
        t_p = jax.lax.fori_loop(0, nch, phase_b, topk_init())[1]
    else:
        jax.lax.fori_loop(0, nch, phase_b, jnp.int32(0))
        t_p = jax.lax.fori_loop(
            0, KSEL, lambda i, c: topk_step(p_ref[...], c), topk_init())[1]

    ge = jnp.where(lr >= t_p, 1.0, 0.0)
    discrete_ref[...] = ge
    c_full = jnp.sum(ge, axis=1, keepdims=True)
    inexact = jnp.logical_not(jnp.all(c_full == np.float32(KSEL)))

    @pl.when(inexact)
    def _fallback():
        thr = jax.lax.fori_loop(
            0, KSEL, lambda i, c: topk_step(lr, c), topk_init())[1]
        discrete_ref[...] = jnp.where(lr >= thr, 1.0, 0.0)


def _build(B, K, d, interpret=False):
    return pl.pallas_call(
        _block_body,
        grid=(B // RB,),
        in_specs=[
            pl.BlockSpec((RB, d), lambda i: (i, 0)),
            pl.BlockSpec((RB, K, d), lambda i: (i, 0, 0)),
        ],
        out_specs=[
            pl.BlockSpec((RB, d), lambda i: (i, 0)),
            pl.BlockSpec((RB, d), lambda i: (i, 0)),
        ],
        out_shape=[
            jax.ShapeDtypeStruct((B, d), jnp.float32),
            jax.ShapeDtypeStruct((B, d), jnp.float32),
        ],
        scratch_shapes=[
            pltpu.VMEM((RB, K, d), jnp.float32),
            pltpu.VMEM((RB, min(4096, d)), jnp.float32),
        ],
        compiler_params=pltpu.CompilerParams(
            dimension_semantics=("arbitrary",)),
        interpret=interpret,
    )


def kernel(logits, uniform):
    B, d = logits.shape
    K = uniform.shape[1]
    samples, discrete = _build(B, K, d)(logits, uniform)
    return samples, discrete


# speedup vs baseline: 14.0215x; 1.1442x over previous
import functools
import numpy as np
import jax
import jax.numpy as jnp
from jax import lax
from jax.experimental import pallas as pl
from jax.experimental.pallas import tpu as pltpu
from jax.experimental.pallas import tpu_sc as plsc

KSEL = 8
RB = 8
CA = np.float32(10.0 * np.log2(np.e))
CB = np.float32(-10.0 * np.log2(np.log(2.0)))
C10 = np.float32(10.0)

SC_NC = 2
SC_NS = 16
SC_L = 16


def _tree(op, items):
    while len(items) > 1:
        nxt = [op(items[i], items[i + 1]) for i in range(0, len(items) - 1, 2)]
        if len(items) % 2:
            nxt.append(items[-1])
        items = nxt
    return items[0]


def _tc_body(logits_ref, u_ref, samples_ref, z_ref):
    _, kk, d = z_ref.shape
    ch = min(4096, d)
    nch = d // ch
    nsl = ch // 128
    tiny = np.float32(np.finfo(np.float32).tiny)
    neg_inf = np.float32(-np.inf)

    def phase_a(j, carry):
        ms, ss = carry
        js = pl.ds(j * ch, ch)
        new_ms, new_ss = [], []
        for r in range(RB):
            u = u_ref[r, :, js]
            wp = -jnp.log2(jnp.maximum(u, tiny))
            lp = CA * logits_ref[pl.ds(r, 1), js] + CB
            z = lp - C10 * jnp.log2(wp)
            z_ref[r, :, js] = z
            sl = [z[:, s * 128:(s + 1) * 128] for s in range(nsl)]
            cm = _tree(jnp.maximum, sl)
            mn = jnp.maximum(ms[r], cm)
            se = _tree(jnp.add, [jnp.exp2(x - mn) for x in sl])
            new_ss.append(ss[r] * jnp.exp2(ms[r] - mn) + se)
            new_ms.append(mn)
        return tuple(new_ms), tuple(new_ss)

    init_m = tuple(jnp.full((kk, 128), neg_inf, jnp.float32)
                   for _ in range(RB))
    init_s = tuple(jnp.zeros((kk, 128), jnp.float32) for _ in range(RB))
    ms, ss = jax.lax.fori_loop(0, nch, phase_a, (init_m, init_s))

    cs = []
    for r in range(RB):
        m = jnp.max(ms[r], axis=1, keepdims=True)
        s = jnp.sum(ss[r] * jnp.exp2(ms[r] - m), axis=1, keepdims=True)
        cs.append(m + jnp.log2(s))

    def phase_b(j, carry):
        js = pl.ds(j * ch, ch)
        rows = []
        for r in range(RB):
            y = z_ref[r, :, js] - cs[r]
            rows.append(jnp.max(y, axis=0, keepdims=True))
        stacked = jnp.concatenate(rows, axis=0)
        samples_ref[:, js] = jnp.exp2(stacked)
        return carry

    jax.lax.fori_loop(0, nch, phase_b, jnp.int32(0))


def _tc_build(B, K, d, interpret=False):
    return pl.pallas_call(
        _tc_body,
        grid=(B // RB,),
        in_specs=[
            pl.BlockSpec((RB, d), lambda i: (i, 0)),
            pl.BlockSpec((RB, K, d), lambda i: (i, 0, 0)),
        ],
        out_specs=pl.BlockSpec((RB, d), lambda i: (i, 0)),
        out_shape=jax.ShapeDtypeStruct((B, d), jnp.float32),
        scratch_shapes=[pltpu.VMEM((RB, K, d), jnp.float32)],
        compiler_params=pltpu.CompilerParams(
            dimension_semantics=("arbitrary",)),
        interpret=interpret,
    )


SC_S = 4


def _sc_body(logits_hbm, out_hbm, row_v, cand_v, shf_v):
    B, D = logits_hbm.shape
    L = SC_L
    S = SC_S
    nch = D // L
    npb = nch // S
    neg_inf = np.float32(-np.inf)
    wid = lax.axis_index("s") * 2 + lax.axis_index("c")
    for rr in range(B // 32):
        row = wid * (B // 32) + rr
        pltpu.sync_copy(logits_hbm.at[row], row_v)

        def bub(i, ts):
            out = []
            for s in range(S):
                x = row_v[pl.ds((i * S + s) * L, L)]
                cur = []
                for t in ts[s]:
                    hi = jnp.maximum(t, x)
                    x = jnp.minimum(t, x)
                    cur.append(hi)
                out.append(tuple(cur))
            return tuple(out)
        init = tuple(tuple(jnp.full((L,), neg_inf, jnp.float32)
                           for _ in range(KSEL)) for _ in range(S))
        ts = lax.fori_loop(0, npb, bub, init)

        idx = 0
        for s in range(S):
            for t in ts[s]:
                cand_v[pl.ds(idx * L, L)] = t
                idx += 1
        ncand = S * KSEL

        def redmax(v):
            m = v
            for sh in (8, 4, 2, 1):
                shf_v[pl.ds(0, L)] = m
                m = jnp.maximum(m, shf_v[pl.ds(sh, L)])
            return m[0]

        def redsum(v):
            m = v
            for sh in (8, 4, 2, 1):
                shf_v[pl.ds(0, L)] = m
                m = m + shf_v[pl.ds(sh, L)]
            return m[0]

        shf_v[pl.ds(L, L)] = jnp.full((L,), neg_inf, jnp.float32)

        def sel_iter(it, carry):
            t, thr, cum, done = carry
            m = jnp.full((L,), neg_inf, jnp.float32)
            for c in range(ncand):
                x = cand_v[pl.ds(c * L, L)]
                m = jnp.maximum(m, jnp.where(x < t, x, neg_inf))
            mx = redmax(m)
            cnt = jnp.zeros((L,), jnp.float32)
            for c in range(ncand):
                x = cand_v[pl.ds(c * L, L)]
                cnt = cnt + jnp.where(x == mx, 1.0, 0.0)
            shf_v[pl.ds(L, L)] = jnp.zeros((L,), jnp.float32)
            cum = cum + redsum(cnt)
            shf_v[pl.ds(L, L)] = jnp.full((L,), neg_inf, jnp.float32)
            hit = jnp.where(cum >= np.float32(KSEL), 1.0, 0.0)
            newly = hit * (1.0 - done)
            thr = jnp.where(newly > 0.0, mx, thr)
            done = jnp.maximum(done, hit)
            return (mx, thr, cum, done)

        init_s = (jnp.float32(np.inf), jnp.float32(0.0), jnp.float32(0.0),
                  jnp.float32(0.0))
        thr = lax.fori_loop(0, KSEL, sel_iter, init_s)[1]

        def p3(i, carry):
            js = pl.ds(i * L, L)
            row_v[js] = jnp.where(row_v[js] >= thr, 1.0, 0.0)
            return carry
        lax.fori_loop(0, nch, p3, jnp.int32(0))
        pltpu.sync_copy(row_v, out_hbm.at[row])


def _sc_build(B, d):
    mesh = plsc.VectorSubcoreMesh(core_axis_name="c", subcore_axis_name="s",
                                  num_cores=SC_NC, num_subcores=SC_NS)
    return functools.partial(
        pl.kernel,
        out_type=jax.ShapeDtypeStruct((B, d), jnp.float32),
        mesh=mesh,
        scratch_types=[
            pltpu.VMEM((d,), jnp.float32),
            pltpu.VMEM((SC_S * KSEL * SC_L,), jnp.float32),
            pltpu.VMEM((2 * SC_L,), jnp.float32),
        ],
    )(_sc_body)


def kernel(logits, uniform):
    B, d = logits.shape
    K = uniform.shape[1]
    discrete = _sc_build(B, d)(logits)
    samples = _tc_build(B, K, d)(logits, uniform)
    return samples, discrete
